# Initial kernel scaffold; baseline (speedup 1.0000x reference)
import jax
import jax.numpy as jnp
from jax.experimental import pallas as pl


def kernel(edge_index, edge_type, emb, w1, root1, b1, w2, root2, b2):
    raise NotImplementedError("write your pallas kernel here")


# speedup vs baseline: 16.7580x; 16.7580x over previous
import functools

import jax
import jax.numpy as jnp
from jax import lax
from jax.experimental import pallas as pl
from jax.experimental.pallas import tpu as pltpu
from jax.experimental.pallas import tpu_sc as plsc

N_NODES = 10000
N_R = 16
D = 128
E = 320000

NC = 2
NS = 16
L = 16
NW = NC * NS

PAD_DST = N_NODES
NODES_P = 10112
RPS = NODES_P // NS
CN = NODES_P * N_R
CNS = CN // NS
EP = 327680
EPW = EP // NW
BLK = 128
NBLK = EPW // BLK

_mesh = plsc.VectorSubcoreMesh(core_axis_name="c", subcore_axis_name="s")

_GDN = lax.GatherDimensionNumbers(
    offset_dims=(), collapsed_slice_dims=(0,), start_index_map=(0,))


def _dg(v, idx):
    return lax.gather(v, idx[:, None], _GDN, (1,),
                      mode=lax.GatherScatterMode.PROMISE_IN_BOUNDS)


def _splat(v, i):
    return _dg(v, jnp.full((L,), i, jnp.int32))


@functools.partial(
    pl.kernel,
    out_type=jax.ShapeDtypeStruct((NC * CN,), jnp.float32),
    mesh=_mesh,
    scratch_types=[
        pltpu.VMEM((BLK,), jnp.int32),
        pltpu.VMEM((BLK,), jnp.int32),
        pltpu.VMEM((BLK,), jnp.int32),
        pltpu.VMEM((BLK,), jnp.float32),
        pltpu.VMEM_SHARED((CN,), jnp.float32),
    ],
)
def _k_count(dst_hbm, t_hbm, zc_hbm, cnt_hbm, dst_v, t_v, idx_v, ones_v, cnt_sh):
    cid = lax.axis_index("c")
    sid = lax.axis_index("s")
    wid = sid * NC + cid

    for q in range(BLK // L):
        ones_v[pl.ds(q * L, L)] = jnp.full((L,), 1.0, jnp.float32)
    pltpu.sync_copy(zc_hbm.at[pl.ds(sid * CNS, CNS)],
                    cnt_sh.at[pl.ds(sid * CNS, CNS)])
    plsc.subcore_barrier()

    base = wid * EPW

    @pl.loop(0, NBLK)
    def _(blk):
        off = base + blk * BLK
        pltpu.sync_copy(t_hbm.at[pl.ds(off, BLK)], t_v)
        pltpu.sync_copy(dst_hbm.at[pl.ds(off, BLK)], dst_v)
        for q in range(BLK // L):
            sl = pl.ds(q * L, L)
            idx_v[sl] = dst_v[sl] * N_R + t_v[sl]
        pltpu.sync_copy(ones_v, cnt_sh.at[idx_v], add=True)

    plsc.subcore_barrier()
    pltpu.sync_copy(cnt_sh.at[pl.ds(sid * CNS, CNS)],
                    cnt_hbm.at[pl.ds(cid * CN + sid * CNS, CNS)])


@functools.partial(
    pl.kernel,
    out_type=[jax.ShapeDtypeStruct((EP,), jnp.int32),
              jax.ShapeDtypeStruct((EP,), jnp.float32)],
    mesh=_mesh,
    scratch_types=[
        pltpu.VMEM((BLK,), jnp.int32),
        pltpu.VMEM((BLK,), jnp.int32),
        pltpu.VMEM((BLK,), jnp.int32),
        pltpu.VMEM((BLK,), jnp.int32),
        pltpu.VMEM((BLK,), jnp.int32),
        pltpu.VMEM((BLK,), jnp.int32),
        pltpu.VMEM((BLK,), jnp.float32),
        pltpu.VMEM((BLK,), jnp.float32),
        pltpu.VMEM((BLK,), jnp.float32),
        pltpu.SemaphoreType.DMA,
        pltpu.SemaphoreType.DMA,
    ],
)
def _k_scale(src_hbm, dst_hbm, t_hbm, cnt_hbm, g_hbm, sc_hbm,
             src_v, dst_v, t_v, g_v, i0_v, i1_v, c0_v, c1_v, sc_v, sem0, sem1):
    cid = lax.axis_index("c")
    sid = lax.axis_index("s")
    wid = sid * NC + cid
    base = wid * EPW

    @pl.loop(0, NBLK)
    def _(blk):
        off = base + blk * BLK
        pltpu.sync_copy(src_hbm.at[pl.ds(off, BLK)], src_v)
        pltpu.sync_copy(dst_hbm.at[pl.ds(off, BLK)], dst_v)
        pltpu.sync_copy(t_hbm.at[pl.ds(off, BLK)], t_v)
        for q in range(BLK // L):
            sl = pl.ds(q * L, L)
            i0 = dst_v[sl] * N_R + t_v[sl]
            i0_v[sl] = i0
            i1_v[sl] = i0 + CN
            g_v[sl] = t_v[sl] * N_NODES + src_v[sl]
        cp0 = pltpu.async_copy(cnt_hbm.at[i0_v], c0_v, sem0)
        cp1 = pltpu.async_copy(cnt_hbm.at[i1_v], c1_v, sem1)
        cp0.wait()
        cp1.wait()
        for q in range(BLK // L):
            sl = pl.ds(q * L, L)
            sc_v[sl] = 1.0 / jnp.maximum(c0_v[sl] + c1_v[sl], 1.0)
        pltpu.sync_copy(g_v, g_hbm.at[pl.ds(off, BLK)])
        pltpu.sync_copy(sc_v, sc_hbm.at[pl.ds(off, BLK)])


@functools.partial(
    pl.kernel,
    out_type=jax.ShapeDtypeStruct((NC, NODES_P, D), jnp.float32),
    mesh=_mesh,
    scratch_types=[
        pltpu.VMEM((BLK,), jnp.int32),
        pltpu.VMEM((BLK,), jnp.int32),
        pltpu.VMEM((BLK,), jnp.float32),
        pltpu.VMEM((BLK, D), jnp.float32),
        pltpu.VMEM_SHARED((NODES_P, D), jnp.float32),
        pltpu.SemaphoreType.DMA,
    ],
)
def _k_agg(y_hbm, g_hbm, dst_hbm, sc_hbm, z_hbm, acc_hbm,
           g_v, dst_v, sc_v, rows_v, acc_sh, sem):
    cid = lax.axis_index("c")
    sid = lax.axis_index("s")
    wid = sid * NC + cid
    base = wid * EPW

    pltpu.sync_copy(z_hbm.at[pl.ds(sid * RPS, RPS)],
                    acc_sh.at[pl.ds(sid * RPS, RPS)])
    plsc.subcore_barrier()

    @pl.loop(0, NBLK)
    def _(blk):
        off = base + blk * BLK
        pltpu.sync_copy(g_hbm.at[pl.ds(off, BLK)], g_v)
        pltpu.sync_copy(dst_hbm.at[pl.ds(off, BLK)], dst_v)
        pltpu.sync_copy(sc_hbm.at[pl.ds(off, BLK)], sc_v)
        pltpu.async_copy(y_hbm.at[g_v], rows_v, sem).wait()
        for g in range(BLK // L):
            sch = sc_v[pl.ds(g * L, L)]
            for i in range(L):
                e = g * L + i
                s = _splat(sch, i)
                for c in range(D // L):
                    sl = pl.ds(c * L, L)
                    rows_v[e, sl] = rows_v[e, sl] * s
        pltpu.sync_copy(rows_v, acc_sh.at[dst_v], add=True)

    plsc.subcore_barrier()
    pltpu.sync_copy(acc_sh.at[pl.ds(sid * RPS, RPS)],
                    acc_hbm.at[cid].at[pl.ds(sid * RPS, RPS)])


BJ = 1000
NBJ = N_NODES // BJ


def _tc_layer(x, adds, w, root, b):
    combine = adds is not None

    def body(*refs):
        if combine:
            x_ref, a0_ref, a1_ref, w_ref, root_ref, b_ref, y_ref, o_ref = refs
        else:
            x_ref, w_ref, root_ref, b_ref, y_ref, o_ref = refs
        r = pl.program_id(1)
        xb = x_ref[...]
        if combine:
            xb = jnp.maximum(xb + a0_ref[...] + a1_ref[...], 0.0)
        y_ref[0] = lax.dot_general(xb, w_ref[0], (((1,), (0,)), ((), ())),
                                   precision=lax.Precision.HIGHEST)

        @pl.when(r == 0)
        def _():
            o_ref[...] = lax.dot_general(
                xb, root_ref[...], (((1,), (0,)), ((), ())),
                precision=lax.Precision.HIGHEST) + b_ref[...]

    x_spec = pl.BlockSpec((BJ, D), lambda j, r: (j, 0))
    in_specs = [x_spec]
    args = [x]
    if combine:
        in_specs += [x_spec, x_spec]
        args += [adds[0], adds[1]]
    in_specs += [
        pl.BlockSpec((1, D, D), lambda j, r: (r, 0, 0)),
        pl.BlockSpec((D, D), lambda j, r: (0, 0)),
        pl.BlockSpec((1, D), lambda j, r: (0, 0)),
    ]
    args += [w, root, b.reshape(1, D)]
    return pl.pallas_call(
        body,
        grid=(NBJ, N_R),
        in_specs=in_specs,
        out_specs=[
            pl.BlockSpec((1, BJ, D), lambda j, r: (r, j, 0)),
            pl.BlockSpec((BJ, D), lambda j, r: (j, 0)),
        ],
        out_shape=[
            jax.ShapeDtypeStruct((N_R, N_NODES, D), jnp.float32),
            jax.ShapeDtypeStruct((N_NODES, D), jnp.float32),
        ],
    )(*args)


def _tc_combine(o, a0, a1):
    def body(o_ref, a0_ref, a1_ref, out_ref):
        out_ref[...] = o_ref[...] + a0_ref[...] + a1_ref[...]

    spec = pl.BlockSpec((BJ, D), lambda j: (j, 0))
    return pl.pallas_call(
        body,
        grid=(NBJ,),
        in_specs=[spec, spec, spec],
        out_specs=spec,
        out_shape=jax.ShapeDtypeStruct((N_NODES, D), jnp.float32),
    )(o, a0, a1)


def kernel(edge_index, edge_type, emb, w1, root1, b1, w2, root2, b2):
    src = edge_index[0]
    dst = edge_index[1]
    pad = EP - E
    srcp = jnp.concatenate([src, jnp.zeros((pad,), jnp.int32)])
    dstp = jnp.concatenate([dst, jnp.full((pad,), PAD_DST, jnp.int32)])
    tp = jnp.concatenate([edge_type, jnp.zeros((pad,), jnp.int32)])

    zc = jnp.zeros((CN,), jnp.float32)
    cnt = _k_count(dstp, tp, zc)
    g, scale = _k_scale(srcp, dstp, tp, cnt)
    zacc = jnp.zeros((NODES_P, D), jnp.float32)

    y1, o1 = _tc_layer(emb, None, w1, root1, b1)
    a1 = _k_agg(y1.reshape(N_R * N_NODES, D), g, dstp, scale, zacc)
    y2, o2 = _tc_layer(o1, (a1[0, :N_NODES], a1[1, :N_NODES]), w2, root2, b2)
    a2 = _k_agg(y2.reshape(N_R * N_NODES, D), g, dstp, scale, zacc)
    return _tc_combine(o2, a2[0, :N_NODES], a2[1, :N_NODES])
